# baseline (speedup 1.0000x reference)
import functools

import jax
import jax.numpy as jnp
from jax import lax
from jax.experimental import pallas as pl
from jax.experimental.pallas import tpu as pltpu
from jax.experimental.pallas import tpu_sc as plsc

N = 10000
E = 320000
D = 128

NC = 2
NS = 16
NW = NC * NS
K = 128
EW = 10240
EP = NW * EW
CHUNKS = EW // K
NP = 10240
RPT = NP // NS
LANES = 16

_mesh = plsc.VectorSubcoreMesh(core_axis_name="c", subcore_axis_name="s")


@functools.partial(
    pl.kernel,
    mesh=_mesh,
    out_type=jax.ShapeDtypeStruct((NC, NP, D), jnp.float32),
    scratch_types=[
        pltpu.VMEM_SHARED((NP, D), jnp.float32),
        pltpu.VMEM((CHUNKS, K), jnp.int32),
        pltpu.VMEM((K,), jnp.int32),
        pltpu.VMEM((K,), jnp.int32),
        pltpu.VMEM((K,), jnp.float32),
        pltpu.VMEM((K,), jnp.float32),
        pltpu.VMEM((K, D), jnp.float32),
        pltpu.VMEM((K, D), jnp.float32),
        pltpu.SemaphoreType.DMA,
        pltpu.SemaphoreType.DMA,
    ],
)
def _spmm(x_hbm, src_hbm, dst_hbm, w_hbm, zero_hbm, out_hbm,
          acc_sh, src_all, dst0, dst1, w0, w1, rows0, rows1, sem0, sem1):
    c = lax.axis_index("c")
    s = lax.axis_index("s")
    wid = s * NC + c

    pltpu.sync_copy(zero_hbm, acc_sh.at[pl.ds(s * RPT, RPT)])
    pltpu.sync_copy(src_hbm.at[wid], src_all)
    plsc.subcore_barrier()

    bufs = (rows0, rows1)
    dsts = (dst0, dst1)
    ws = (w0, w1)
    sems = (sem0, sem1)

    def gather_start(i, b):
        pltpu.make_async_copy(dst_hbm.at[wid, i], dsts[b], sems[b]).start()
        pltpu.make_async_copy(w_hbm.at[wid, i], ws[b], sems[b]).start()
        pltpu.make_async_copy(x_hbm.at[src_all.at[i]], bufs[b], sems[b]).start()

    def gather_wait(i, b):
        pltpu.make_async_copy(dst_hbm.at[wid, i], dsts[b], sems[b]).wait()
        pltpu.make_async_copy(w_hbm.at[wid, i], ws[b], sems[b]).wait()
        pltpu.make_async_copy(x_hbm.at[src_all.at[i]], bufs[b], sems[b]).wait()

    def scale_scatter(i, b):
        rows_v = bufs[b]
        w_v = ws[b]

        def scale_body(g, _):
            wv = w_v[pl.ds(g * LANES, LANES)]
            for e16 in range(LANES):
                wspl = jnp.broadcast_to(
                    lax.slice(wv, (e16,), (e16 + 1,)), (LANES,))
                e = g * LANES + e16
                for j in range(D // LANES):
                    sl = rows_v[e, pl.ds(j * LANES, LANES)]
                    rows_v[e, pl.ds(j * LANES, LANES)] = sl * wspl
            return 0

        lax.fori_loop(0, K // LANES, scale_body, 0)
        pltpu.sync_copy(rows_v, acc_sh.at[dsts[b]], add=True)

    gather_start(0, 0)

    def chunk_body(i2, _):
        i = 2 * i2
        gather_start(i + 1, 1)
        gather_wait(i, 0)
        scale_scatter(i, 0)

        @pl.when(i2 < CHUNKS // 2 - 1)
        def _():
            gather_start(i + 2, 0)

        gather_wait(i + 1, 1)
        scale_scatter(i + 1, 1)
        return 0

    lax.fori_loop(0, CHUNKS // 2, chunk_body, 0)
    plsc.subcore_barrier()
    pltpu.sync_copy(acc_sh.at[pl.ds(s * RPT, RPT)],
                    out_hbm.at[c, pl.ds(s * RPT, RPT)])


@functools.partial(
    pl.kernel,
    mesh=_mesh,
    out_type=jax.ShapeDtypeStruct((NW, LANES), jnp.float32),
    scratch_types=[pltpu.VMEM((LANES,), jnp.float32)],
)
def _sc_warmup(a_hbm, out_hbm, buf_v):
    c = lax.axis_index("c")
    s = lax.axis_index("s")
    wid = s * NC + c
    pltpu.sync_copy(a_hbm, buf_v)
    pltpu.sync_copy(buf_v, out_hbm.at[wid])


_BR = 1000


def _add_body(p_ref, o_ref):
    o_ref[...] = p_ref[0] + p_ref[1]


_add = pl.pallas_call(
    _add_body,
    grid=(N // _BR,),
    in_specs=[pl.BlockSpec((NC, _BR, D), lambda i: (0, i, 0))],
    out_specs=pl.BlockSpec((_BR, D), lambda i: (i, 0)),
    out_shape=jax.ShapeDtypeStruct((NP, D), jnp.float32),
)


def _combo_body(delta_ref, x_ref, t1_ref, q_ref, low_ref, band_ref, high_ref):
    d = delta_ref[0]
    t0 = x_ref[...]
    t1 = t1_ref[...]
    t2 = q_ref[0] + q_ref[1]
    low_ref[...] = t2 + (-2.0 * d - 2.0) * t1 + (d + 1.0) * (d + 1.0) * t0
    band_ref[...] = 2.0 * (-t2 + (2.0 * d + 1.0) * t1 - (d * d + d) * t0)
    high_ref[...] = t2 - 2.0 * d * t1 + d * d * t0


_out_nd = jax.ShapeDtypeStruct((N, D), jnp.float32)
_combo = pl.pallas_call(
    _combo_body,
    grid=(N // _BR,),
    in_specs=[
        pl.BlockSpec(memory_space=pltpu.SMEM),
        pl.BlockSpec((_BR, D), lambda i: (i, 0)),
        pl.BlockSpec((_BR, D), lambda i: (i, 0)),
        pl.BlockSpec((NC, _BR, D), lambda i: (0, i, 0)),
    ],
    out_specs=[pl.BlockSpec((_BR, D), lambda i: (i, 0))] * 3,
    out_shape=[_out_nd, _out_nd, _out_nd],
)


def kernel(x, edge_index, edge_weight, delta):
    pad = EP - E
    src_p = jnp.concatenate([edge_index[0], jnp.zeros((pad,), jnp.int32)])
    src_p = src_p.reshape(NW, CHUNKS, K)
    dst_p = jnp.concatenate([edge_index[1], jnp.zeros((pad,), jnp.int32)])
    dst_p = dst_p.reshape(NW, CHUNKS, K)
    w_p = jnp.concatenate([edge_weight, jnp.zeros((pad,), jnp.float32)])
    w_p = w_p.reshape(NW, CHUNKS, K)
    zero = jnp.zeros((RPT, D), jnp.float32)

    wu = _sc_warmup(zero[0, :LANES])
    p = _spmm(x + wu[0, 0] * 0.0, src_p, dst_p, w_p, zero)
    tx1 = _add(p)
    q = _spmm(tx1, src_p, dst_p, w_p, zero)
    low, band, high = _combo(delta, x, tx1, q)
    return (low, band, high)


# speedup vs baseline: 1.3093x; 1.3093x over previous
import functools

import jax
import jax.numpy as jnp
from jax import lax
from jax.experimental import pallas as pl
from jax.experimental.pallas import tpu as pltpu
from jax.experimental.pallas import tpu_sc as plsc

N = 10000
E = 320000
D = 128

NC = 2
NS = 16
NW = NC * NS
K = 128
EW = 10240
EP = NW * EW
CHUNKS = EW // K
NP = 10240
RPT = NP // NS
LANES = 16

_mesh = plsc.VectorSubcoreMesh(core_axis_name="c", subcore_axis_name="s")


@functools.partial(
    pl.kernel,
    mesh=_mesh,
    out_type=jax.ShapeDtypeStruct((NC, NP, D), jnp.float32),
    scratch_types=[
        pltpu.VMEM_SHARED((NP, D), jnp.float32),
        pltpu.VMEM((CHUNKS, K), jnp.int32),
        pltpu.VMEM((K,), jnp.int32),
        pltpu.VMEM((K,), jnp.int32),
        pltpu.VMEM((K,), jnp.float32),
        pltpu.VMEM((K,), jnp.float32),
        pltpu.VMEM((K, D), jnp.float32),
        pltpu.VMEM((K, D), jnp.float32),
        pltpu.SemaphoreType.DMA,
        pltpu.SemaphoreType.DMA,
    ],
)
def _spmm(x_hbm, src_hbm, dst_hbm, w_hbm, zero_hbm, out_hbm,
          acc_sh, src_all, dst0, dst1, w0, w1, rows0, rows1, sem0, sem1):
    c = lax.axis_index("c")
    s = lax.axis_index("s")
    wid = s * NC + c

    pltpu.sync_copy(zero_hbm, acc_sh.at[pl.ds(s * RPT, RPT)])
    pltpu.sync_copy(src_hbm.at[wid], src_all)
    plsc.subcore_barrier()

    bufs = (rows0, rows1)
    dsts = (dst0, dst1)
    ws = (w0, w1)
    sems = (sem0, sem1)

    def gather_start(i, b):
        pltpu.make_async_copy(dst_hbm.at[wid, i], dsts[b], sems[b]).start()
        pltpu.make_async_copy(w_hbm.at[wid, i], ws[b], sems[b]).start()
        pltpu.make_async_copy(x_hbm.at[src_all.at[i]], bufs[b], sems[b]).start()

    def gather_wait(i, b):
        pltpu.make_async_copy(dst_hbm.at[wid, i], dsts[b], sems[b]).wait()
        pltpu.make_async_copy(w_hbm.at[wid, i], ws[b], sems[b]).wait()
        pltpu.make_async_copy(x_hbm.at[src_all.at[i]], bufs[b], sems[b]).wait()

    def scale_scatter(i, b):
        rows_v = bufs[b]
        w_v = ws[b]

        def scale_body(g, _):
            wv = w_v[pl.ds(g * LANES, LANES)]
            for e16 in range(LANES):
                wspl = jnp.broadcast_to(
                    lax.slice(wv, (e16,), (e16 + 1,)), (LANES,))
                e = g * LANES + e16
                for j in range(D // LANES):
                    sl = rows_v[e, pl.ds(j * LANES, LANES)]
                    rows_v[e, pl.ds(j * LANES, LANES)] = sl * wspl
            return 0

        lax.fori_loop(0, K // LANES, scale_body, 0)
        pltpu.sync_copy(rows_v, acc_sh.at[dsts[b]], add=True)

    gather_start(0, 0)

    def chunk_body(i2, _):
        i = 2 * i2
        gather_start(i + 1, 1)
        gather_wait(i, 0)
        scale_scatter(i, 0)

        @pl.when(i2 < CHUNKS // 2 - 1)
        def _():
            gather_start(i + 2, 0)

        gather_wait(i + 1, 1)
        scale_scatter(i + 1, 1)
        return 0

    lax.fori_loop(0, CHUNKS // 2, chunk_body, 0)
    plsc.subcore_barrier()
    pltpu.sync_copy(acc_sh.at[pl.ds(s * RPT, RPT)],
                    out_hbm.at[c, pl.ds(s * RPT, RPT)])


@functools.partial(
    pl.kernel,
    mesh=_mesh,
    out_type=jax.ShapeDtypeStruct((NW, LANES), jnp.float32),
    scratch_types=[pltpu.VMEM((LANES,), jnp.float32)],
)
def _sc_warmup(a_hbm, out_hbm, buf_v):
    c = lax.axis_index("c")
    s = lax.axis_index("s")
    wid = s * NC + c
    pltpu.sync_copy(a_hbm, buf_v)
    pltpu.sync_copy(buf_v, out_hbm.at[wid])


_BR = 1000


def _add_body(p_ref, o_ref):
    o_ref[...] = p_ref[0] + p_ref[1]


_add = pl.pallas_call(
    _add_body,
    grid=(N // _BR,),
    in_specs=[pl.BlockSpec((NC, _BR, D), lambda i: (0, i, 0))],
    out_specs=pl.BlockSpec((_BR, D), lambda i: (i, 0)),
    out_shape=jax.ShapeDtypeStruct((NP, D), jnp.float32),
)


def _combo_body(delta_ref, x_ref, t1_ref, q_ref, low_ref, band_ref, high_ref):
    d = delta_ref[0]
    t0 = x_ref[...]
    t1 = t1_ref[...]
    t2 = q_ref[0] + q_ref[1]
    low_ref[...] = t2 + (-2.0 * d - 2.0) * t1 + (d + 1.0) * (d + 1.0) * t0
    band_ref[...] = 2.0 * (-t2 + (2.0 * d + 1.0) * t1 - (d * d + d) * t0)
    high_ref[...] = t2 - 2.0 * d * t1 + d * d * t0


_out_nd = jax.ShapeDtypeStruct((N, D), jnp.float32)
_combo = pl.pallas_call(
    _combo_body,
    grid=(N // _BR,),
    in_specs=[
        pl.BlockSpec(memory_space=pltpu.SMEM),
        pl.BlockSpec((_BR, D), lambda i: (i, 0)),
        pl.BlockSpec((_BR, D), lambda i: (i, 0)),
        pl.BlockSpec((NC, _BR, D), lambda i: (0, i, 0)),
    ],
    out_specs=[pl.BlockSpec((_BR, D), lambda i: (i, 0))] * 3,
    out_shape=[_out_nd, _out_nd, _out_nd],
)


def kernel(x, edge_index, edge_weight, delta):
    pad = EP - E
    src_p = jnp.concatenate([edge_index[0], jnp.zeros((pad,), jnp.int32)])
    src_p = src_p.reshape(NW, CHUNKS, K)
    dst_p = jnp.concatenate([edge_index[1], jnp.zeros((pad,), jnp.int32)])
    dst_p = dst_p.reshape(NW, CHUNKS, K)
    w_p = jnp.concatenate([edge_weight, jnp.zeros((pad,), jnp.float32)])
    w_p = w_p.reshape(NW, CHUNKS, K)
    zero = jnp.zeros((RPT, D), jnp.float32)

    x_p = jnp.concatenate([x, jnp.zeros((NP - N, D), jnp.float32)])
    p = _spmm(x_p, src_p, dst_p, w_p, zero)
    tx1 = _add(p)
    q = _spmm(tx1, src_p, dst_p, w_p, zero)
    low, band, high = _combo(delta, x, tx1, q)
    return (low, band, high)
